# baseline (speedup 1.0000x reference)
import functools

import jax
import jax.numpy as jnp
from jax import lax
from jax.experimental import pallas as pl
from jax.experimental.pallas import tpu as pltpu
from jax.experimental.pallas import tpu_sc as plsc

_B, _HIS, _D = 64, 50, 256
_N = 100000
_L = 6
_H, _DK = 16, 16
_HID = 200

_NW = 32
_NCH = 16
_CHAINS = _B * _HIS
_NCHUNK = _CHAINS // _NCH
_UNROLL = 4


def _chain_kernel_body(table, click, nbr, news, x_out, mask_out,
                       cur_v, cand_v, nbr_v, clk_v, cidx_v, idx0_v,
                       selT_v, gidx_v, gmask_v, xrows_v,
                       sem_c, sem_n, sem_x):
    core = lax.axis_index("c")
    sub = lax.axis_index("s")
    wid = sub * 2 + core

    lane = lax.iota(jnp.int32, _NCH)
    lane3 = [lane * 3 + j for j in range(3)]
    nchunks = jnp.where(wid < (_NCHUNK % _NW), _NCHUNK // _NW + 1, _NCHUNK // _NW)

    def chunk_body(kk, _):
        base = (wid + kk * _NW) * _NCH
        pltpu.sync_copy(click.at[pl.ds(base, _NCH)], clk_v)
        pltpu.sync_copy(news.at[pl.ds(base, _NCH), :], cur_v)
        idx = clk_v[...]

        for t in range(_L):
            valid = idx > 0
            idx0 = jnp.clip(idx, 1, _N) - 1
            idx0_v[...] = idx0
            for j in range(3):
                plsc.store_scatter(cidx_v, [lane3[j]], idx0 * 3 + j)
            ccand = pltpu.async_copy(table.at[cidx_v], cand_v, sem_c)
            cnbr = pltpu.async_copy(nbr.at[cidx_v], nbr_v, sem_n)
            ccand.wait()
            cnbr.wait()

            def dbody(i, carry):
                s0, s1, s2 = carry
                for u in range(_UNROLL):
                    dcol = jnp.full((_NCH,), i * _UNROLL + u, jnp.int32)
                    cu = plsc.load_gather(cur_v, [lane, dcol])
                    c0 = plsc.load_gather(cand_v, [lane3[0], dcol])
                    c1 = plsc.load_gather(cand_v, [lane3[1], dcol])
                    c2 = plsc.load_gather(cand_v, [lane3[2], dcol])
                    s0 = s0 + c0 * cu
                    s1 = s1 + c1 * cu
                    s2 = s2 + c2 * cu
                return s0, s1, s2

            zero = jnp.zeros((_NCH,), jnp.float32)
            s0, s1, s2 = lax.fori_loop(0, _D // _UNROLL, dbody, (zero, zero, zero))
            s0 = jnp.where(valid, s0, 0.0)
            s1 = jnp.where(valid, s1, 0.0)
            s2 = jnp.where(valid, s2, 0.0)
            m01 = jnp.maximum(s0, s1)
            maxv = jnp.maximum(m01, s2)
            mi = jnp.where(s1 > s0, 1, 0)
            mi = jnp.where(s2 > m01, 2, mi)
            nz = maxv != 0.0
            nxt = plsc.load_gather(nbr_v, [lane * 3 + mi])
            idx = jnp.where(nz, nxt, idx)
            seli = jnp.where(nz, idx0 * 3 + mi, -1)
            plsc.store_scatter(selT_v, [lane * _L + t], seli)

        for g in range(_L):
            v = selT_v[pl.ds(g * _NCH, _NCH)]
            gidx_v[pl.ds(g * _NCH, _NCH)] = jnp.maximum(v, 0)
            mk = jnp.where(v >= 0, 1.0, 0.0)
            rr = lane + g * _NCH
            for col in range(8):
                plsc.store_scatter(gmask_v, [rr, jnp.full((_NCH,), col, jnp.int32)], mk)

        pltpu.async_copy(table.at[gidx_v], xrows_v, sem_x).wait()
        pltpu.sync_copy(xrows_v, x_out.at[pl.ds(base * _L, _NCH * _L), :])
        pltpu.sync_copy(gmask_v, mask_out.at[pl.ds(base * _L, _NCH * _L), :])
        return 0

    lax.fori_loop(0, nchunks, chunk_body, 0)


def _make_chain_kernel():
    mesh = plsc.VectorSubcoreMesh(core_axis_name="c", subcore_axis_name="s")
    return pl.kernel(
        _chain_kernel_body,
        out_type=[
            jax.ShapeDtypeStruct((_CHAINS * _L, _D), jnp.float32),
            jax.ShapeDtypeStruct((_CHAINS * _L, 8), jnp.float32),
        ],
        mesh=mesh,
        scratch_types=[
            pltpu.VMEM((_NCH, _D), jnp.float32),
            pltpu.VMEM((_NCH * 3, _D), jnp.float32),
            pltpu.VMEM((_NCH * 3,), jnp.int32),
            pltpu.VMEM((_NCH,), jnp.int32),
            pltpu.VMEM((_NCH * 3,), jnp.int32),
            pltpu.VMEM((_NCH,), jnp.int32),
            pltpu.VMEM((_NCH * _L,), jnp.int32),
            pltpu.VMEM((_NCH * _L,), jnp.int32),
            pltpu.VMEM((_NCH * _L, 8), jnp.float32),
            pltpu.VMEM((_NCH * _L, _D), jnp.float32),
            pltpu.SemaphoreType.DMA,
            pltpu.SemaphoreType.DMA,
            pltpu.SemaphoreType.DMA,
        ],
        compiler_params=pltpu.CompilerParams(needs_layout_passes=False),
    )


def _encoder_body(x_ref, m_ref, wq_ref, bq_ref, wk_ref, bk_ref, wv_ref, bv_ref,
                  g1_ref, b1l_ref, w1_ref, bb1_ref, w2_ref, b2_ref,
                  g2_ref, b2l_ref, out_ref, o_scr):
    x = x_ref[0]
    mcol = m_ref[0][:, :1]
    xm = x * mcol
    f32 = jnp.float32
    q = jnp.dot(xm, wq_ref[...], preferred_element_type=f32) + bq_ref[...]
    k = jnp.dot(xm, wk_ref[...], preferred_element_type=f32) + bk_ref[...]
    v = jnp.dot(xm, wv_ref[...], preferred_element_type=f32) + bv_ref[...]
    scale = 1.0 / jnp.sqrt(jnp.float32(_DK))
    for h in range(_H):
        sl = slice(h * _DK, (h + 1) * _DK)
        qh, kh, vh = q[:, sl], k[:, sl], v[:, sl]
        s = lax.dot_general(qh, kh, (((1,), (1,)), ((), ())),
                            preferred_element_type=f32) * scale
        s = s - jnp.max(s, axis=-1, keepdims=True)
        p = jnp.exp(s)
        p = p / jnp.sum(p, axis=-1, keepdims=True)
        o_scr[:, sl] = jnp.dot(p, vh, preferred_element_type=f32)
    o = o_scr[...]
    mu = jnp.mean(o, axis=-1, keepdims=True)
    xc = o - mu
    var = jnp.mean(xc * xc, axis=-1, keepdims=True)
    x1 = xc / jnp.sqrt(var + 1e-5) * g1_ref[...] + b1l_ref[...]
    t1 = jnp.tanh(jnp.dot(x1, w1_ref[...], preferred_element_type=f32) + bb1_ref[...])
    e = jnp.sum(t1 * w2_ref[...], axis=1, keepdims=True) + b2_ref[...]
    eT = jnp.transpose(e)
    riota = lax.broadcasted_iota(jnp.int32, (_HIS, _HIS * _L), 0)
    ciota = lax.broadcasted_iota(jnp.int32, (_HIS, _HIS * _L), 1)
    seg = (ciota // _L) == riota
    sm = jnp.where(seg, jnp.broadcast_to(eT, (_HIS, _HIS * _L)), -1e30)
    mg = jnp.max(sm, axis=1, keepdims=True)
    P = jnp.where(seg, jnp.exp(sm - mg), 0.0)
    Wm = P / jnp.sum(P, axis=1, keepdims=True)
    pooled = jnp.dot(Wm, x1, preferred_element_type=f32)
    mu2 = jnp.mean(pooled, axis=-1, keepdims=True)
    pc = pooled - mu2
    var2 = jnp.mean(pc * pc, axis=-1, keepdims=True)
    out_ref[0] = pc / jnp.sqrt(var2 + 1e-5) * g2_ref[...] + b2l_ref[...]


def _full(shape):
    return pl.BlockSpec(shape, lambda b: tuple(0 for _ in shape))


def kernel(news_input, click_history, outputs_dict, neighbors, Wq, bq, Wk, bk,
           Wv, bv, ln1_g, ln1_b, w1, b1, w2, b2, ln2_g, ln2_b):
    table = outputs_dict.reshape(_N * 3, _D)
    click = click_history.reshape(_CHAINS)
    news = news_input.reshape(_CHAINS, _D)
    nbr_flat = neighbors.reshape(_N * 3)

    x_flat, mask_flat = _make_chain_kernel()(table, click, nbr_flat, news)
    x = x_flat.reshape(_B, _HIS * _L, _D)
    mask = mask_flat.reshape(_B, _HIS * _L, 8)

    S = _HIS * _L
    out = pl.pallas_call(
        _encoder_body,
        grid=(_B,),
        in_specs=[
            pl.BlockSpec((1, S, _D), lambda b: (b, 0, 0)),
            pl.BlockSpec((1, S, 8), lambda b: (b, 0, 0)),
            _full((_D, _D)), _full((1, _D)),
            _full((_D, _D)), _full((1, _D)),
            _full((_D, _D)), _full((1, _D)),
            _full((1, _D)), _full((1, _D)),
            _full((_D, _HID)), _full((1, _HID)),
            _full((1, _HID)), _full((1, 1)),
            _full((1, _D)), _full((1, _D)),
        ],
        out_specs=pl.BlockSpec((1, _HIS, _D), lambda b: (b, 0, 0)),
        out_shape=jax.ShapeDtypeStruct((_B, _HIS, _D), jnp.float32),
        scratch_shapes=[pltpu.VMEM((S, _D), jnp.float32)],
    )(
        x, mask,
        Wq, bq.reshape(1, _D), Wk, bk.reshape(1, _D), Wv, bv.reshape(1, _D),
        ln1_g.reshape(1, _D), ln1_b.reshape(1, _D),
        w1, b1.reshape(1, _HID), w2.reshape(1, _HID), b2.reshape(1, 1),
        ln2_g.reshape(1, _D), ln2_b.reshape(1, _D),
    )
    return out


# speedup vs baseline: 1.4156x; 1.4156x over previous
import functools

import jax
import jax.numpy as jnp
from jax import lax
from jax.experimental import pallas as pl
from jax.experimental.pallas import tpu as pltpu
from jax.experimental.pallas import tpu_sc as plsc

_B, _HIS, _D = 64, 50, 256
_N = 100000
_L = 6
_H, _DK = 16, 16
_HID = 200

_NW = 32
_NCH = 16
_CHAINS = _B * _HIS
_NCHUNK = _CHAINS // _NCH
_UNROLL = 4
_BB = 8


def _chain_kernel_body(table, click, nbr, news, x3_out, w8_out,
                       cur_v, cand_v, nbr_v, clk_v, cidx_v, idx0_v,
                       w8_v, sem_c, sem_n, sem_w):
    core = lax.axis_index("c")
    sub = lax.axis_index("s")
    wid = sub * 2 + core

    lane = lax.iota(jnp.int32, _NCH)
    lane3 = [lane * 3 + j for j in range(3)]
    nchunks = jnp.where(wid < (_NCHUNK % _NW), _NCHUNK // _NW + 1, _NCHUNK // _NW)

    def chunk_body(kk, _):
        base = (wid + kk * _NW) * _NCH
        pltpu.sync_copy(click.at[pl.ds(base, _NCH)], clk_v)
        pltpu.sync_copy(news.at[pl.ds(base, _NCH), :], cur_v)
        idx = clk_v[...]

        pend = []
        for t in range(_L):
            valid = idx > 0
            idx0 = jnp.clip(idx, 1, _N) - 1
            idx0_v[...] = idx0
            for j in range(3):
                plsc.store_scatter(cidx_v, [lane3[j]], idx0 * 3 + j)
            while len(pend) > 1:
                pend.pop(0).wait()
            buf = t % 2
            ccand = pltpu.async_copy(table.at[idx0_v], cand_v.at[buf], sem_c)
            cnbr = pltpu.async_copy(nbr.at[cidx_v], nbr_v, sem_n)
            ccand.wait()
            cnbr.wait()
            wr = pltpu.make_async_copy(
                cand_v.at[buf], x3_out.at[t, pl.ds(base, _NCH), :], sem_w)
            wr.start()
            pend.append(wr)

            bsel = jnp.full((_NCH,), buf, jnp.int32)
            off = [jnp.full((_NCH,), j * _D, jnp.int32) for j in range(3)]

            def dbody(i, carry):
                s0, s1, s2 = carry
                for u in range(_UNROLL):
                    dcol = jnp.full((_NCH,), i * _UNROLL + u, jnp.int32)
                    cu = plsc.load_gather(cur_v, [lane, dcol])
                    c0 = plsc.load_gather(cand_v, [bsel, lane, off[0] + dcol])
                    c1 = plsc.load_gather(cand_v, [bsel, lane, off[1] + dcol])
                    c2 = plsc.load_gather(cand_v, [bsel, lane, off[2] + dcol])
                    s0 = s0 + c0 * cu
                    s1 = s1 + c1 * cu
                    s2 = s2 + c2 * cu
                return s0, s1, s2

            zero = jnp.zeros((_NCH,), jnp.float32)
            s0, s1, s2 = lax.fori_loop(0, _D // _UNROLL, dbody, (zero, zero, zero))
            s0 = jnp.where(valid, s0, 0.0)
            s1 = jnp.where(valid, s1, 0.0)
            s2 = jnp.where(valid, s2, 0.0)
            m01 = jnp.maximum(s0, s1)
            maxv = jnp.maximum(m01, s2)
            mi = jnp.where(s1 > s0, 1, 0)
            mi = jnp.where(s2 > m01, 2, mi)
            nz = maxv != 0.0
            nxt = plsc.load_gather(nbr_v, [lane * 3 + mi])
            idx = jnp.where(nz, nxt, idx)
            for j in range(3):
                wv = jnp.where((mi == j) & nz, 1.0, 0.0)
                plsc.store_scatter(w8_v, [lane, jnp.full((_NCH,), j, jnp.int32)], wv)
            pltpu.sync_copy(w8_v, w8_out.at[t, pl.ds(base, _NCH), :])

        for p in pend:
            p.wait()
        return 0

    lax.fori_loop(0, nchunks, chunk_body, 0)


def _make_chain_kernel():
    mesh = plsc.VectorSubcoreMesh(core_axis_name="c", subcore_axis_name="s")
    return pl.kernel(
        _chain_kernel_body,
        out_type=[
            jax.ShapeDtypeStruct((_L, _CHAINS, 3 * _D), jnp.float32),
            jax.ShapeDtypeStruct((_L, _CHAINS, 8), jnp.float32),
        ],
        mesh=mesh,
        scratch_types=[
            pltpu.VMEM((_NCH, _D), jnp.float32),
            pltpu.VMEM((2, _NCH, 3 * _D), jnp.float32),
            pltpu.VMEM((_NCH * 3,), jnp.int32),
            pltpu.VMEM((_NCH,), jnp.int32),
            pltpu.VMEM((_NCH * 3,), jnp.int32),
            pltpu.VMEM((_NCH,), jnp.int32),
            pltpu.VMEM((_NCH, 8), jnp.float32),
            pltpu.SemaphoreType.DMA,
            pltpu.SemaphoreType.DMA,
            pltpu.SemaphoreType.DMA,
        ],
        compiler_params=pltpu.CompilerParams(needs_layout_passes=False),
    )


def _encoder_body(c0, c1, c2, c3r, c4, c5, w0, w1r, w2r, w3r, w4r, w5r,
                  wq_ref, bq_ref, wk_ref, bk_ref, wv_ref, bv_ref,
                  g1_ref, b1l_ref, mw1_ref, mb1_ref, mw2_ref, mb2_ref,
                  g2_ref, b2l_ref, out_ref, o_scr):
    f32 = jnp.float32
    S = _HIS * _L
    G = _BB * _HIS
    c_refs = [c0, c1, c2, c3r, c4, c5]
    w_refs = [w0, w1r, w2r, w3r, w4r, w5r]
    scale = 1.0 / jnp.sqrt(jnp.float32(_DK))
    riota = lax.broadcasted_iota(jnp.int32, (_HIS, S), 0)
    ciota = lax.broadcasted_iota(jnp.int32, (_HIS, S), 1)
    seg = (ciota % _HIS) == riota

    xsel = []
    for t in range(_L):
        ct = c_refs[t][...]
        wt = w_refs[t][...]
        acc = ct[:, 0:_D] * wt[:, 0:1]
        acc = acc + ct[:, _D:2 * _D] * wt[:, 1:2]
        acc = acc + ct[:, 2 * _D:3 * _D] * wt[:, 2:3]
        xsel.append(acc)

    outs = []
    for sub in range(_BB):
        xm = jnp.concatenate(
            [xsel[t][sub * _HIS:(sub + 1) * _HIS] for t in range(_L)], axis=0)
        q = jnp.dot(xm, wq_ref[...], preferred_element_type=f32) + bq_ref[...]
        k = jnp.dot(xm, wk_ref[...], preferred_element_type=f32) + bk_ref[...]
        v = jnp.dot(xm, wv_ref[...], preferred_element_type=f32) + bv_ref[...]
        for h in range(_H):
            sl = slice(h * _DK, (h + 1) * _DK)
            qh, kh, vh = q[:, sl], k[:, sl], v[:, sl]
            s = lax.dot_general(qh, kh, (((1,), (1,)), ((), ())),
                                preferred_element_type=f32) * scale
            s = s - jnp.max(s, axis=-1, keepdims=True)
            p = jnp.exp(s)
            p = p / jnp.sum(p, axis=-1, keepdims=True)
            o_scr[:, sl] = jnp.dot(p, vh, preferred_element_type=f32)
        o = o_scr[...]
        mu = jnp.mean(o, axis=-1, keepdims=True)
        xc = o - mu
        var = jnp.mean(xc * xc, axis=-1, keepdims=True)
        x1 = xc / jnp.sqrt(var + 1e-5) * g1_ref[...] + b1l_ref[...]
        t1 = jnp.tanh(jnp.dot(x1, mw1_ref[...], preferred_element_type=f32)
                      + mb1_ref[...])
        e = jnp.sum(t1 * mw2_ref[...], axis=1, keepdims=True) + mb2_ref[...]
        eT = jnp.transpose(e)
        sm = jnp.where(seg, jnp.broadcast_to(eT, (_HIS, S)), -1e30)
        mg = jnp.max(sm, axis=1, keepdims=True)
        P = jnp.where(seg, jnp.exp(sm - mg), 0.0)
        Wm = P / jnp.sum(P, axis=1, keepdims=True)
        pooled = jnp.dot(Wm, x1, preferred_element_type=f32)
        mu2 = jnp.mean(pooled, axis=-1, keepdims=True)
        pc = pooled - mu2
        var2 = jnp.mean(pc * pc, axis=-1, keepdims=True)
        outs.append(pc / jnp.sqrt(var2 + 1e-5) * g2_ref[...] + b2l_ref[...])
    out_ref[...] = jnp.concatenate(outs, axis=0)


def _full(shape):
    return pl.BlockSpec(shape, lambda b: tuple(0 for _ in shape))


def _mk_block(t, shape):
    return pl.BlockSpec(shape, lambda b, tt=t: ((_B // _BB) * tt + b, 0))


def kernel(news_input, click_history, outputs_dict, neighbors, Wq, bq, Wk, bk,
           Wv, bv, ln1_g, ln1_b, w1, b1, w2, b2, ln2_g, ln2_b):
    click = click_history.reshape(_CHAINS)
    news = news_input.reshape(_CHAINS, _D)
    table = outputs_dict.reshape(_N, 3 * _D)
    nbr_flat = neighbors.reshape(_N * 3)

    x3, w8 = _make_chain_kernel()(table, click, nbr_flat, news)
    x3v = x3.reshape(_L * _CHAINS, 3 * _D)
    w8v = w8.reshape(_L * _CHAINS, 8)
    G = _BB * _HIS

    out2 = pl.pallas_call(
        _encoder_body,
        grid=(_B // _BB,),
        in_specs=(
            [_mk_block(t, (G, 3 * _D)) for t in range(_L)]
            + [_mk_block(t, (G, 8)) for t in range(_L)]
            + [
                _full((_D, _D)), _full((1, _D)),
                _full((_D, _D)), _full((1, _D)),
                _full((_D, _D)), _full((1, _D)),
                _full((1, _D)), _full((1, _D)),
                _full((_D, _HID)), _full((1, _HID)),
                _full((1, _HID)), _full((1, 1)),
                _full((1, _D)), _full((1, _D)),
            ]
        ),
        out_specs=pl.BlockSpec((G, _D), lambda b: (b, 0)),
        out_shape=jax.ShapeDtypeStruct((_CHAINS, _D), jnp.float32),
        scratch_shapes=[pltpu.VMEM((_HIS * _L, _D), jnp.float32)],
    )(
        x3v, x3v, x3v, x3v, x3v, x3v,
        w8v, w8v, w8v, w8v, w8v, w8v,
        Wq, bq.reshape(1, _D), Wk, bk.reshape(1, _D), Wv, bv.reshape(1, _D),
        ln1_g.reshape(1, _D), ln1_b.reshape(1, _D),
        w1, b1.reshape(1, _HID), w2.reshape(1, _HID), b2.reshape(1, 1),
        ln2_g.reshape(1, _D), ln2_b.reshape(1, _D),
    )
    return out2.reshape(_B, _HIS, _D)
